# Initial kernel scaffold; baseline (speedup 1.0000x reference)
import jax
import jax.numpy as jnp
from jax.experimental import pallas as pl


def kernel(x, edge_attr, pos, force_vector, material_params, params, edge_index, batch):
    raise NotImplementedError("write your pallas kernel here")


# speedup vs baseline: 2.6292x; 2.6292x over previous
import functools

import jax
import jax.numpy as jnp
from jax import lax
from jax.experimental import pallas as pl
from jax.experimental.pallas import tpu as pltpu
from jax.experimental.pallas import tpu_sc as plsc

_N = 10000
_E = 320000
_B = 8
_ND = 12
_ED = 6
_H = 128
_NPAD = 10240

_NCORES = 2
_NSUB = 16
_K = 80
_EPC = _E // _NCORES
_EPS = _EPC // _NSUB
_CHUNKS = _EPS // _K
_RW = _NPAD // _NSUB

_F32 = jnp.float32


def _full_spec(shape):
    return pl.BlockSpec(shape, lambda i: (0,) * len(shape))


def _edge_enc_body(ea_ref, w1, b1, w2, b2, g, be,
                   l0w, l0b, l1w, l1b, l2w, l2b, o0, o1, o2):
    t = jnp.dot(ea_ref[...], w1[...], preferred_element_type=_F32) + b1[...]
    t = jnp.maximum(t, 0.0)
    t = jnp.dot(t, w2[...], preferred_element_type=_F32) + b2[...]
    m = jnp.mean(t, axis=-1, keepdims=True)
    d = t - m
    var = jnp.mean(d * d, axis=-1, keepdims=True)
    t = d * lax.rsqrt(var + 1e-5) * g[...] + be[...]
    o0[...] = jnp.dot(t, l0w[...], preferred_element_type=_F32) + l0b[...]
    o1[...] = jnp.dot(t, l1w[...], preferred_element_type=_F32) + l1b[...]
    o2[...] = jnp.dot(t, l2w[...], preferred_element_type=_F32) + l2b[...]


_BE = 3200


def _edge_encode(edge_attr, *weights):
    specs = [pl.BlockSpec((_BE, _ED), lambda i: (i, 0))]
    specs += [_full_spec(w.shape) for w in weights]
    out_spec = pl.BlockSpec((_BE, _H), lambda i: (i, 0))
    return pl.pallas_call(
        _edge_enc_body,
        grid=(_E // _BE,),
        in_specs=specs,
        out_specs=[out_spec] * 3,
        out_shape=[jax.ShapeDtypeStruct((_E, _H), _F32)] * 3,
    )(edge_attr, *weights)


def _node_enc_body(x_ref, w1, b1, w2, b2, g, be, out):
    t = jnp.dot(x_ref[...], w1[...], preferred_element_type=_F32) + b1[...]
    t = jnp.maximum(t, 0.0)
    t = jnp.dot(t, w2[...], preferred_element_type=_F32) + b2[...]
    m = jnp.mean(t, axis=-1, keepdims=True)
    d = t - m
    var = jnp.mean(d * d, axis=-1, keepdims=True)
    out[...] = d * lax.rsqrt(var + 1e-5) * g[...] + be[...]


def _node_encode(x, *weights):
    return pl.pallas_call(
        _node_enc_body,
        out_shape=jax.ShapeDtypeStruct((_N, _H), _F32),
    )(x, *weights)


def _node_update_body(h_ref, agg_ref, w1, b1, w2, b2, g, be, out):
    h = h_ref[...]
    agg = agg_ref[0, 0:_N, :] + agg_ref[1, 0:_N, :]
    z = h + agg
    z = jnp.maximum(jnp.dot(z, w1[...], preferred_element_type=_F32) + b1[...], 0.0)
    z = jnp.dot(z, w2[...], preferred_element_type=_F32) + b2[...]
    z = jnp.maximum(z, 0.0)
    y = h + z
    m = jnp.mean(y, axis=-1, keepdims=True)
    d = y - m
    var = jnp.mean(d * d, axis=-1, keepdims=True)
    out[...] = d * lax.rsqrt(var + 1e-5) * g[...] + be[...]


def _node_update(h, aggp, *weights):
    return pl.pallas_call(
        _node_update_body,
        out_shape=jax.ShapeDtypeStruct((_N, _H), _F32),
    )(h, aggp, *weights)


def _graph_scale_body(pos_ref, batch_ref, fvT, mpT, w1t, b1c, w2t, b2s, lb_ref,
                      dsg_out, mean_out):
    ids = lax.broadcasted_iota(jnp.int32, (1, _B), 1)
    mask = batch_ref[...] == ids
    z3 = pos_ref[:, 2:3]
    zmin = jnp.min(jnp.where(mask, z3, jnp.inf), axis=0, keepdims=True)
    zmax = jnp.max(jnp.where(mask, z3, -jnp.inf), axis=0, keepdims=True)
    geom = jnp.maximum(zmax - zmin, 1e-6)
    fm = jnp.sqrt(jnp.sum(fvT[...] * fvT[...], axis=0, keepdims=True))
    emod = jnp.maximum(mpT[0:1, :], 1.0)
    nu = jnp.clip(mpT[1:2, :], 0.0, 0.49)
    feats = jnp.concatenate(
        [jnp.log(fm + 1.0), jnp.log(emod + 1e-12), nu, jnp.log(geom + 1e-6)],
        axis=0)
    a = jnp.maximum(jnp.dot(w1t[...], feats, preferred_element_type=_F32) + b1c[...], 0.0)
    lm = jnp.dot(w2t[...], a, preferred_element_type=_F32) + b2s[...]
    lm = jnp.clip(lm, -8.0, 8.0)
    base = 1e-5 + jnp.log(1.0 + jnp.exp(lb_ref[...]))
    dsg = jnp.maximum(base * jnp.exp(lm), 1e-5)
    dsg_out[...] = dsg
    mean_out[...] = jnp.mean(dsg, axis=-1, keepdims=True)


def _graph_scale(pos, batch_col, fvT, mpT, w1t, b1c, w2t, b2s, lb):
    return pl.pallas_call(
        _graph_scale_body,
        out_shape=[jax.ShapeDtypeStruct((1, _B), _F32),
                   jax.ShapeDtypeStruct((1, 1), _F32)],
    )(pos, batch_col, fvT, mpT, w1t, b1c, w2t, b2s, lb)


def _decoder_body(h_ref, dsg_col, batch_ref, dw1, db1, dw2, db2,
                  sw1, sb1, sw2, sb2,
                  u_out, ru_out, s_out, ls_out, sf_out):
    h = h_ref[...]
    t = jnp.maximum(jnp.dot(h, dw1[...], preferred_element_type=_F32) + db1[...], 0.0)
    raw_u = jnp.dot(t, dw2[...], preferred_element_type=_F32) + db2[...]
    t2 = jnp.maximum(jnp.dot(h, sw1[...], preferred_element_type=_F32) + sb1[...], 0.0)
    log_s = jnp.dot(t2, sw2[...], preferred_element_type=_F32) + sb2[...]
    log_s = jnp.clip(log_s, 0.0, 30.0)
    s = jnp.exp(log_s)
    ids = lax.broadcasted_iota(jnp.int32, (1, _B), 1)
    onehot = (batch_ref[...] == ids).astype(_F32)
    dn = jnp.dot(onehot, dsg_col[...], preferred_element_type=_F32)
    u_out[...] = raw_u * dn
    ru_out[...] = raw_u
    s_out[...] = s
    ls_out[...] = log_s
    sf_out[...] = 2.5e8 / (s + 1e-8)


def _decoder(h, dsg_col, batch_col, *weights):
    return pl.pallas_call(
        _decoder_body,
        out_shape=[jax.ShapeDtypeStruct((_N, 3), _F32),
                   jax.ShapeDtypeStruct((_N, 3), _F32),
                   jax.ShapeDtypeStruct((_N, 1), _F32),
                   jax.ShapeDtypeStruct((_N, 1), _F32),
                   jax.ShapeDtypeStruct((_N, 1), _F32)],
    )(h, dsg_col, batch_col, *weights)


def _sc_agg_body(h_hbm, ea_hbm, src_hbm, dst_hbm, out_hbm,
                 src_v, dst_v, rows_v, acc_sh, sem):
    c = lax.axis_index("c")
    s = lax.axis_index("s")

    def _zero(i, _):
        r = i // 8
        col = (i % 8) * 16
        rows_v[r, pl.ds(col, 16)] = jnp.zeros((16,), _F32)
        return 0
    lax.fori_loop(0, _K * 8, _zero, 0)

    for j in range(_RW // _K):
        pltpu.sync_copy(rows_v, acc_sh.at[pl.ds(s * _RW + j * _K, _K)])
    plsc.subcore_barrier()

    def _chunk(t, _):
        base = c * _EPC + s * _EPS + t * _K
        pltpu.sync_copy(src_hbm.at[pl.ds(base, _K)], src_v)
        pltpu.sync_copy(dst_hbm.at[pl.ds(base, _K)], dst_v)
        pltpu.sync_copy(ea_hbm.at[pl.ds(base, _K)], rows_v)
        pltpu.async_copy(h_hbm.at[src_v], rows_v, sem, add=True).wait()

        def _relu(r, _):
            for j in range(8):
                sl = pl.ds(j * 16, 16)
                rows_v[r, sl] = jnp.maximum(rows_v[r, sl], 0.0)
            return 0
        lax.fori_loop(0, _K, _relu, 0)
        pltpu.sync_copy(rows_v, acc_sh.at[dst_v], add=True)
        return 0
    lax.fori_loop(0, _CHUNKS, _chunk, 0)

    plsc.subcore_barrier()
    pltpu.sync_copy(acc_sh.at[pl.ds(s * _RW, _RW)],
                    out_hbm.at[c, pl.ds(s * _RW, _RW)])


@functools.cache
def _sc_agg_fn():
    return pl.kernel(
        _sc_agg_body,
        out_type=jax.ShapeDtypeStruct((_NCORES, _NPAD, _H), _F32),
        mesh=plsc.VectorSubcoreMesh(core_axis_name="c", subcore_axis_name="s",
                                    num_cores=_NCORES, num_subcores=_NSUB),
        scratch_types=[
            pltpu.VMEM((_K,), jnp.int32),
            pltpu.VMEM((_K,), jnp.int32),
            pltpu.VMEM((_K, _H), _F32),
            pltpu.VMEM_SHARED((_NPAD, _H), _F32),
            pltpu.SemaphoreType.DMA,
        ],
    )


def _sc_agg(h, ea_l, src, dst):
    return _sc_agg_fn()(h, ea_l, src, dst)


def kernel(x, edge_attr, pos, force_vector, material_params, params, edge_index, batch):
    p = params
    row = lambda v: v.reshape(1, -1)
    src = edge_index[0]
    dst = edge_index[1]
    batch_col = batch.reshape(_N, 1).astype(jnp.int32)

    ea0, ea1, ea2 = _edge_encode(
        edge_attr,
        p['ee_W1'], row(p['ee_b1']), p['ee_W2'], row(p['ee_b2']),
        row(p['ee_g']), row(p['ee_be']),
        p['c0_lin_W'], row(p['c0_lin_b']),
        p['c1_lin_W'], row(p['c1_lin_b']),
        p['c2_lin_W'], row(p['c2_lin_b']))

    h = _node_encode(
        x, p['ne_W1'], row(p['ne_b1']), p['ne_W2'], row(p['ne_b2']),
        row(p['ne_g']), row(p['ne_be']))

    for i, ea_l in enumerate((ea0, ea1, ea2)):
        aggp = _sc_agg(h, ea_l, src, dst)
        h = _node_update(
            h, aggp,
            p[f'c{i}_m_W1'], row(p[f'c{i}_m_b1']),
            p[f'c{i}_m_W2'], row(p[f'c{i}_m_b2']),
            row(p[f'pn{i}_g']), row(p[f'pn{i}_be']))

    dsgT, meanv = _graph_scale(
        pos, batch_col,
        force_vector.T, material_params.T,
        p['sm_W1'].T, p['sm_b1'].reshape(-1, 1),
        p['sm_W2'].T, p['sm_b2'].reshape(1, 1),
        p['lb'].reshape(1, 1))
    dsg_col = dsgT.reshape(_B, 1)

    u, raw_u, s, log_s, safety = _decoder(
        h, dsg_col, batch_col,
        p['dh_W1'], row(p['dh_b1']), p['dh_W2'], row(p['dh_b2']),
        p['sh_W1'], row(p['sh_b1']), p['sh_W2'], row(p['sh_b2']))

    return (u, raw_u, s, log_s, meanv.reshape(()), dsg_col, safety)


# speedup vs baseline: 5.1039x; 1.9413x over previous
import functools

import jax
import jax.numpy as jnp
from jax import lax
from jax.experimental import pallas as pl
from jax.experimental.pallas import tpu as pltpu
from jax.experimental.pallas import tpu_sc as plsc

_N = 10000
_E = 320000
_B = 8
_ND = 12
_ED = 6
_H = 128
_NPAD = 10240

_NCORES = 2
_NSUB = 16
_K = 80
_EPC = _E // _NCORES
_EPS = _EPC // _NSUB
_CHUNKS = _EPS // _K
_RW = _NPAD // _NSUB

_F32 = jnp.float32


def _full_spec(shape):
    return pl.BlockSpec(shape, lambda i: (0,) * len(shape))


def _edge_enc_body(ea_ref, w1, b1, w2, b2, g, be,
                   l0w, l0b, l1w, l1b, l2w, l2b, o0, o1, o2):
    t = jnp.dot(ea_ref[...], w1[...], preferred_element_type=_F32) + b1[...]
    t = jnp.maximum(t, 0.0)
    t = jnp.dot(t, w2[...], preferred_element_type=_F32) + b2[...]
    m = jnp.mean(t, axis=-1, keepdims=True)
    d = t - m
    var = jnp.mean(d * d, axis=-1, keepdims=True)
    t = d * lax.rsqrt(var + 1e-5) * g[...] + be[...]
    o0[...] = jnp.dot(t, l0w[...], preferred_element_type=_F32) + l0b[...]
    o1[...] = jnp.dot(t, l1w[...], preferred_element_type=_F32) + l1b[...]
    o2[...] = jnp.dot(t, l2w[...], preferred_element_type=_F32) + l2b[...]


_BE = 3200


def _edge_encode(edge_attr, *weights):
    specs = [pl.BlockSpec((_BE, _ED), lambda i: (i, 0))]
    specs += [_full_spec(w.shape) for w in weights]
    out_spec = pl.BlockSpec((_BE, _H), lambda i: (i, 0))
    return pl.pallas_call(
        _edge_enc_body,
        grid=(_E // _BE,),
        in_specs=specs,
        out_specs=[out_spec] * 3,
        out_shape=[jax.ShapeDtypeStruct((_E, _H), _F32)] * 3,
    )(edge_attr, *weights)


def _node_enc_body(x_ref, w1, b1, w2, b2, g, be, out):
    t = jnp.dot(x_ref[...], w1[...], preferred_element_type=_F32) + b1[...]
    t = jnp.maximum(t, 0.0)
    t = jnp.dot(t, w2[...], preferred_element_type=_F32) + b2[...]
    m = jnp.mean(t, axis=-1, keepdims=True)
    d = t - m
    var = jnp.mean(d * d, axis=-1, keepdims=True)
    out[...] = d * lax.rsqrt(var + 1e-5) * g[...] + be[...]


def _node_encode(x, *weights):
    return pl.pallas_call(
        _node_enc_body,
        out_shape=jax.ShapeDtypeStruct((_N, _H), _F32),
    )(x, *weights)


def _node_update_body(h_ref, agg_ref, w1, b1, w2, b2, g, be, out):
    h = h_ref[...]
    agg = agg_ref[0, 0:_N, :] + agg_ref[1, 0:_N, :]
    z = h + agg
    z = jnp.maximum(jnp.dot(z, w1[...], preferred_element_type=_F32) + b1[...], 0.0)
    z = jnp.dot(z, w2[...], preferred_element_type=_F32) + b2[...]
    z = jnp.maximum(z, 0.0)
    y = h + z
    m = jnp.mean(y, axis=-1, keepdims=True)
    d = y - m
    var = jnp.mean(d * d, axis=-1, keepdims=True)
    out[...] = d * lax.rsqrt(var + 1e-5) * g[...] + be[...]


def _node_update(h, aggp, *weights):
    return pl.pallas_call(
        _node_update_body,
        out_shape=jax.ShapeDtypeStruct((_N, _H), _F32),
    )(h, aggp, *weights)


def _graph_scale_body(pos_ref, batch_ref, fvT, mpT, w1t, b1c, w2t, b2s, lb_ref,
                      dsg_out, mean_out):
    ids = lax.broadcasted_iota(jnp.int32, (1, _B), 1)
    mask = batch_ref[...] == ids
    z3 = pos_ref[:, 2:3]
    zmin = jnp.min(jnp.where(mask, z3, jnp.inf), axis=0, keepdims=True)
    zmax = jnp.max(jnp.where(mask, z3, -jnp.inf), axis=0, keepdims=True)
    geom = jnp.maximum(zmax - zmin, 1e-6)
    fm = jnp.sqrt(jnp.sum(fvT[...] * fvT[...], axis=0, keepdims=True))
    emod = jnp.maximum(mpT[0:1, :], 1.0)
    nu = jnp.clip(mpT[1:2, :], 0.0, 0.49)
    feats = jnp.concatenate(
        [jnp.log(fm + 1.0), jnp.log(emod + 1e-12), nu, jnp.log(geom + 1e-6)],
        axis=0)
    a = jnp.maximum(jnp.dot(w1t[...], feats, preferred_element_type=_F32) + b1c[...], 0.0)
    lm = jnp.dot(w2t[...], a, preferred_element_type=_F32) + b2s[...]
    lm = jnp.clip(lm, -8.0, 8.0)
    base = 1e-5 + jnp.log(1.0 + jnp.exp(lb_ref[...]))
    dsg = jnp.maximum(base * jnp.exp(lm), 1e-5)
    dsg_out[...] = dsg
    mean_out[...] = jnp.mean(dsg, axis=-1, keepdims=True)


def _graph_scale(pos, batch_col, fvT, mpT, w1t, b1c, w2t, b2s, lb):
    return pl.pallas_call(
        _graph_scale_body,
        out_shape=[jax.ShapeDtypeStruct((1, _B), _F32),
                   jax.ShapeDtypeStruct((1, 1), _F32)],
    )(pos, batch_col, fvT, mpT, w1t, b1c, w2t, b2s, lb)


def _decoder_body(h_ref, dsg_col, batch_ref, dw1, db1, dw2, db2,
                  sw1, sb1, sw2, sb2,
                  u_out, ru_out, s_out, ls_out, sf_out):
    h = h_ref[...]
    t = jnp.maximum(jnp.dot(h, dw1[...], preferred_element_type=_F32) + db1[...], 0.0)
    raw_u = jnp.dot(t, dw2[...], preferred_element_type=_F32) + db2[...]
    t2 = jnp.maximum(jnp.dot(h, sw1[...], preferred_element_type=_F32) + sb1[...], 0.0)
    log_s = jnp.dot(t2, sw2[...], preferred_element_type=_F32) + sb2[...]
    log_s = jnp.clip(log_s, 0.0, 30.0)
    s = jnp.exp(log_s)
    ids = lax.broadcasted_iota(jnp.int32, (1, _B), 1)
    onehot = (batch_ref[...] == ids).astype(_F32)
    dn = jnp.dot(onehot, dsg_col[...], preferred_element_type=_F32)
    u_out[...] = raw_u * dn
    ru_out[...] = raw_u
    s_out[...] = s
    ls_out[...] = log_s
    sf_out[...] = 2.5e8 / (s + 1e-8)


def _decoder(h, dsg_col, batch_col, *weights):
    return pl.pallas_call(
        _decoder_body,
        out_shape=[jax.ShapeDtypeStruct((_N, 3), _F32),
                   jax.ShapeDtypeStruct((_N, 3), _F32),
                   jax.ShapeDtypeStruct((_N, 1), _F32),
                   jax.ShapeDtypeStruct((_N, 1), _F32),
                   jax.ShapeDtypeStruct((_N, 1), _F32)],
    )(h, dsg_col, batch_col, *weights)


_CPS = _EPS // _K
_CPC = _EPC // _K


def _sc_agg_body(h_hbm, ea_hbm, src_hbm, dst_hbm, out_hbm,
                 src_f, dst_v0, dst_v1, dst_v2, ea_b,
                 acc_sh, semA0, semA1, semA2, semB0, semB1, semB2):
    c = lax.axis_index("c")
    s = lax.axis_index("s")
    gbase = c * _EPC + s * _EPS
    semA = (semA0, semA1, semA2)
    semB = (semB0, semB1, semB2)
    dsts = (dst_v0, dst_v1, dst_v2)

    def _zero(i, _):
        r = i // 8
        col = (i % 8) * 16
        ea_b[0, r, pl.ds(col, 16)] = jnp.zeros((16,), _F32)
        return 0
    lax.fori_loop(0, _K * 8, _zero, 0)
    for j in range(_RW // _K):
        pltpu.sync_copy(ea_b.at[0], acc_sh.at[pl.ds(s * _RW + j * _K, _K)])

    pltpu.sync_copy(src_hbm.at[pl.ds(gbase, _EPS)], src_f)
    plsc.subcore_barrier()

    def _s1(t, b):
        base = gbase + t * _K
        pltpu.async_copy(dst_hbm.at[pl.ds(base, _K)], dsts[b], semB[b])
        pltpu.async_copy(ea_hbm.at[pl.ds(base, _K)], ea_b.at[b], semA[b])

    def _s2(t, b):
        base = gbase + t * _K
        pltpu.make_async_copy(ea_hbm.at[pl.ds(base, _K)], ea_b.at[b],
                              semA[b]).wait()
        pltpu.async_copy(h_hbm.at[src_f.at[pl.ds(t * _K, _K)]], ea_b.at[b],
                         semB[b], add=True)

    def _s3(t, b):
        base = gbase + t * _K
        pltpu.make_async_copy(dst_hbm.at[pl.ds(base, _K)], dsts[b],
                              semB[b]).wait()
        pltpu.make_async_copy(h_hbm.at[src_f.at[pl.ds(t * _K, _K)]],
                              ea_b.at[b], semB[b]).wait()
        eb = ea_b.at[b]

        @plsc.parallel_loop(0, _K, 1, unroll=2)
        def _rows(r):
            for j in range(8):
                sl = pl.ds(j * 16, 16)
                eb[r, sl] = jnp.maximum(eb[r, sl], 0.0)

        pltpu.sync_copy(ea_b.at[b], acc_sh.at[dsts[b]], add=True)

    _s1(0, 0)
    _s1(1, 1)
    _s2(0, 0)

    def _pipe(i, _):
        t = 3 * i
        _s1(t + 2, 2); _s2(t + 1, 1); _s3(t, 0)
        _s1(t + 3, 0); _s2(t + 2, 2); _s3(t + 1, 1)
        _s1(t + 4, 1); _s2(t + 3, 0); _s3(t + 2, 2)
        return 0
    lax.fori_loop(0, (_CPS - 2) // 3, _pipe, 0)
    _s2(_CPS - 1, (_CPS - 1) % 3)
    _s3(_CPS - 2, (_CPS - 2) % 3)
    _s3(_CPS - 1, (_CPS - 1) % 3)

    plsc.subcore_barrier()
    pltpu.sync_copy(acc_sh.at[pl.ds(s * _RW, _RW)],
                    out_hbm.at[c, pl.ds(s * _RW, _RW)])


@functools.cache
def _sc_agg_fn():
    return pl.kernel(
        _sc_agg_body,
        out_type=jax.ShapeDtypeStruct((_NCORES, _NPAD, _H), _F32),
        mesh=plsc.VectorSubcoreMesh(core_axis_name="c", subcore_axis_name="s",
                                    num_cores=_NCORES, num_subcores=_NSUB),
        scratch_types=[
            pltpu.VMEM((_EPS,), jnp.int32),
            pltpu.VMEM((_K,), jnp.int32),
            pltpu.VMEM((_K,), jnp.int32),
            pltpu.VMEM((_K,), jnp.int32),
            pltpu.VMEM((3, _K, _H), _F32),
            pltpu.VMEM_SHARED((_NPAD, _H), _F32),
            pltpu.SemaphoreType.DMA,
            pltpu.SemaphoreType.DMA,
            pltpu.SemaphoreType.DMA,
            pltpu.SemaphoreType.DMA,
            pltpu.SemaphoreType.DMA,
            pltpu.SemaphoreType.DMA,
        ],
    )


def _sc_agg(h, ea_l, src, dst):
    return _sc_agg_fn()(h, ea_l, src, dst)


def kernel(x, edge_attr, pos, force_vector, material_params, params, edge_index, batch):
    p = params
    row = lambda v: v.reshape(1, -1)
    src = edge_index[0]
    dst = edge_index[1]
    batch_col = batch.reshape(_N, 1).astype(jnp.int32)

    ea0, ea1, ea2 = _edge_encode(
        edge_attr,
        p['ee_W1'], row(p['ee_b1']), p['ee_W2'], row(p['ee_b2']),
        row(p['ee_g']), row(p['ee_be']),
        p['c0_lin_W'], row(p['c0_lin_b']),
        p['c1_lin_W'], row(p['c1_lin_b']),
        p['c2_lin_W'], row(p['c2_lin_b']))

    h = _node_encode(
        x, p['ne_W1'], row(p['ne_b1']), p['ne_W2'], row(p['ne_b2']),
        row(p['ne_g']), row(p['ne_be']))

    for i, ea_l in enumerate((ea0, ea1, ea2)):
        aggp = _sc_agg(h, ea_l, src, dst)
        h = _node_update(
            h, aggp,
            p[f'c{i}_m_W1'], row(p[f'c{i}_m_b1']),
            p[f'c{i}_m_W2'], row(p[f'c{i}_m_b2']),
            row(p[f'pn{i}_g']), row(p[f'pn{i}_be']))

    dsgT, meanv = _graph_scale(
        pos, batch_col,
        force_vector.T, material_params.T,
        p['sm_W1'].T, p['sm_b1'].reshape(-1, 1),
        p['sm_W2'].T, p['sm_b2'].reshape(1, 1),
        p['lb'].reshape(1, 1))
    dsg_col = dsgT.reshape(_B, 1)

    u, raw_u, s, log_s, safety = _decoder(
        h, dsg_col, batch_col,
        p['dh_W1'], row(p['dh_b1']), p['dh_W2'], row(p['dh_b2']),
        p['sh_W1'], row(p['sh_b1']), p['sh_W2'], row(p['sh_b2']))

    return (u, raw_u, s, log_s, meanv.reshape(()), dsg_col, safety)
